# baseline (speedup 1.0000x reference)
import functools

import jax
import jax.numpy as jnp
from jax import lax
from jax.experimental import pallas as pl
from jax.experimental.pallas import tpu as pltpu
from jax.experimental.pallas import tpu_sc as plsc

_LANES = 16
_SUBCORES = 16
_BR = 32768


def _tc_nll_body(x_ref, lab_ref, nll_ref):
    c = x_ref.shape[1]
    groups = x_ref.shape[0] // 128
    lab = lab_ref[...]
    ones_row = jnp.ones((1, c), jnp.float32)
    for g in range(groups):
        xt = x_ref[pl.ds(g * 128, 128), :].T
        m = jnp.max(xt, axis=0, keepdims=True)
        e = jnp.exp(xt - m)
        s = jnp.dot(ones_row, e, preferred_element_type=jnp.float32)
        onehot = lax.broadcasted_iota(jnp.int32, (c, 128), 0) == lab[g:g + 1, :]
        masked = jnp.where(onehot, xt, 0.0)
        picked = jnp.dot(ones_row, masked, preferred_element_type=jnp.float32)
        nll_ref[pl.ds(g, 1), :] = jnp.log(s) + m - picked


def _sc_scatter_stage(c, rows, lab_off, lab_hbm, loss_hbm, w_hbm,
                      lab_v, loss_v, hist_v, all_v, res_v, w_v, shared):
    wid = lax.axis_index("s")
    base = wid * rows
    pltpu.sync_copy(lab_hbm.at[pl.ds(lab_off + base, rows), :], lab_v)
    pltpu.sync_copy(loss_hbm.at[pl.ds(base, rows), :], loss_v)
    pltpu.sync_copy(w_hbm, w_v)

    zeros = jnp.zeros((_LANES,), jnp.float32)
    for j in range(2 * c // _LANES):
        hist_v[pl.ds(j * _LANES, _LANES)] = zeros
    ones = jnp.ones((_LANES,), jnp.float32)

    def step(r, carry):
        for j in range(128 // _LANES):
            labv = lab_v[r, pl.ds(j * _LANES, _LANES)]
            nllv = loss_v[r, pl.ds(j * _LANES, _LANES)]
            wv = plsc.load_gather(w_v, [labv])
            plsc.addupdate_scatter(hist_v, [labv], wv * nllv)
            plsc.addupdate_scatter(hist_v, [labv + c], ones)
        return carry

    lax.fori_loop(0, rows, step, 0)

    pltpu.sync_copy(hist_v, shared.at[wid])
    plsc.subcore_barrier()

    @pl.when(wid == 0)
    def _():
        pltpu.sync_copy(shared, all_v)
        for j in range(2 * c // _LANES):
            acc = jnp.zeros((_LANES,), jnp.float32)
            for k in range(_SUBCORES):
                acc = acc + all_v[k, pl.ds(j * _LANES, _LANES)]
            res_v[pl.ds(j * _LANES, _LANES)] = acc
    return wid


def _sc_partial_body(num_classes, rows, lab_off,
                     lab_hbm, loss_hbm, w_hbm, part_hbm,
                     lab_v, loss_v, hist_v, all_v, res_v, w_v, shared):
    c = num_classes
    wid = _sc_scatter_stage(c, rows, lab_off, lab_hbm, loss_hbm, w_hbm,
                            lab_v, loss_v, hist_v, all_v, res_v, w_v, shared)

    @pl.when(wid == 0)
    def _():
        pltpu.sync_copy(res_v, part_hbm)


def _sc_final_body(num_classes, rows, lab_off,
                   lab_hbm, loss_hbm, w_hbm, part_hbm, sums_hbm, outw_hbm,
                   lab_v, loss_v, hist_v, all_v, res_v, w_v, part_v, shared):
    c = num_classes
    wid = _sc_scatter_stage(c, rows, lab_off, lab_hbm, loss_hbm, w_hbm,
                            lab_v, loss_v, hist_v, all_v, res_v, w_v, shared)

    @pl.when(wid == 0)
    def _():
        pltpu.sync_copy(part_hbm, part_v)
        for j in range(2 * c // _LANES):
            acc = res_v[pl.ds(j * _LANES, _LANES)] + part_v[pl.ds(j * _LANES, _LANES)]
            res_v[pl.ds(j * _LANES, _LANES)] = acc
        for j in range(c // _LANES):
            cnt = res_v[pl.ds(c + j * _LANES, _LANES)]
            wv = w_v[pl.ds(j * _LANES, _LANES)]
            res_v[pl.ds(c + j * _LANES, _LANES)] = cnt * wv
        pltpu.sync_copy(res_v.at[pl.ds(0, c)], sums_hbm)
        pltpu.sync_copy(res_v.at[pl.ds(c, c)], outw_hbm)


def kernel(inputs, labels, weights):
    n, c = inputs.shape
    lab2d = labels.reshape(n // 128, 128)
    half = n // 2
    hrows = half // 128
    hgrid = half // _BR
    br_rows = _BR // 128

    def tc_half(h):
        boff = h * hgrid
        return pl.pallas_call(
            _tc_nll_body,
            grid=(hgrid,),
            in_specs=[
                pl.BlockSpec((_BR, c), lambda i: (i + boff, 0)),
                pl.BlockSpec((br_rows, 128), lambda i: (i + boff, 0)),
            ],
            out_specs=pl.BlockSpec((br_rows, 128), lambda i: (i, 0)),
            out_shape=jax.ShapeDtypeStruct((hrows, 128), jnp.float32),
        )(inputs, lab2d)

    nll_a = tc_half(0)
    nll_b = tc_half(1)

    rows = hrows // _SUBCORES
    mesh = plsc.VectorSubcoreMesh(
        core_axis_name="c", subcore_axis_name="s", num_cores=1)
    base_scratch = [
        pltpu.VMEM((rows, 128), jnp.int32),
        pltpu.VMEM((rows, 128), jnp.float32),
        pltpu.VMEM((2 * c,), jnp.float32),
        pltpu.VMEM((_SUBCORES, 2 * c), jnp.float32),
        pltpu.VMEM((2 * c,), jnp.float32),
        pltpu.VMEM((c,), jnp.float32),
    ]
    shared_scratch = pltpu.VMEM_SHARED((_SUBCORES, 2 * c), jnp.float32)

    part = pl.kernel(
        functools.partial(_sc_partial_body, c, rows, 0),
        out_type=jax.ShapeDtypeStruct((2 * c,), jnp.float32),
        mesh=mesh,
        scratch_types=base_scratch + [shared_scratch],
        compiler_params=pltpu.CompilerParams(needs_layout_passes=False),
    )(lab2d, nll_a, weights)

    sum_by_class, out_weights = pl.kernel(
        functools.partial(_sc_final_body, c, rows, hrows),
        out_type=(jax.ShapeDtypeStruct((c,), jnp.float32),
                  jax.ShapeDtypeStruct((c,), jnp.float32)),
        mesh=mesh,
        scratch_types=base_scratch
        + [pltpu.VMEM((2 * c,), jnp.float32)]
        + [shared_scratch],
        compiler_params=pltpu.CompilerParams(needs_layout_passes=False),
    )(lab2d, nll_b, weights, part)
    return (sum_by_class, out_weights)


# speedup vs baseline: 1.0403x; 1.0403x over previous
import functools

import jax
import jax.numpy as jnp
from jax import lax
from jax.experimental import pallas as pl
from jax.experimental.pallas import tpu as pltpu
from jax.experimental.pallas import tpu_sc as plsc

_LANES = 16
_SUBCORES = 16
_BR = 32768


def _tc_nll_body(x_ref, lab_ref, nll_ref):
    c = x_ref.shape[1]
    groups = x_ref.shape[0] // 128
    lab = lab_ref[...]
    ones_row = jnp.ones((1, c), jnp.float32)
    for g in range(groups):
        xt = x_ref[pl.ds(g * 128, 128), :].T
        m = jnp.max(xt, axis=0, keepdims=True)
        e = jnp.exp(xt - m)
        s = jnp.dot(ones_row, e, preferred_element_type=jnp.float32)
        onehot = lax.broadcasted_iota(jnp.int32, (c, 128), 0) == lab[g:g + 1, :]
        masked = jnp.where(onehot, xt, 0.0)
        picked = jnp.dot(ones_row, masked, preferred_element_type=jnp.float32)
        nll_ref[pl.ds(g, 1), :] = jnp.log(s) + m - picked


def _sc_scatter_stage(c, rows, lab_off, lab_hbm, loss_hbm, w_hbm,
                      lab_v, loss_v, hist_v, all_v, res_v, w_v, shared):
    wid = lax.axis_index("s")
    base = wid * rows
    pltpu.sync_copy(lab_hbm.at[pl.ds(lab_off + base, rows), :], lab_v)
    pltpu.sync_copy(loss_hbm.at[pl.ds(base, rows), :], loss_v)
    pltpu.sync_copy(w_hbm, w_v)

    zeros = jnp.zeros((_LANES,), jnp.float32)
    for j in range(2 * c // _LANES):
        hist_v[pl.ds(j * _LANES, _LANES)] = zeros
    ones = jnp.ones((_LANES,), jnp.float32)

    def step(r, carry):
        for j in range(128 // _LANES):
            labv = lab_v[r, pl.ds(j * _LANES, _LANES)]
            nllv = loss_v[r, pl.ds(j * _LANES, _LANES)]
            wv = plsc.load_gather(w_v, [labv])
            plsc.addupdate_scatter(hist_v, [labv], wv * nllv)
            plsc.addupdate_scatter(hist_v, [labv + c], ones)
        return carry

    lax.fori_loop(0, rows, step, 0)

    pltpu.sync_copy(hist_v, shared.at[wid])
    plsc.subcore_barrier()

    @pl.when(wid == 0)
    def _():
        pltpu.sync_copy(shared, all_v)
        for j in range(2 * c // _LANES):
            acc = jnp.zeros((_LANES,), jnp.float32)
            for k in range(_SUBCORES):
                acc = acc + all_v[k, pl.ds(j * _LANES, _LANES)]
            res_v[pl.ds(j * _LANES, _LANES)] = acc
    return wid


def _sc_groupby_body(num_classes, rows,
                     lab_hbm, loss_hbm, w_hbm, sums_hbm, outw_hbm,
                     lab_v, loss_v, hist_v, all_v, res_v, w_v, shared):
    c = num_classes
    wid = _sc_scatter_stage(c, rows, 0, lab_hbm, loss_hbm, w_hbm,
                            lab_v, loss_v, hist_v, all_v, res_v, w_v, shared)

    @pl.when(wid == 0)
    def _():
        for j in range(c // _LANES):
            cnt = res_v[pl.ds(c + j * _LANES, _LANES)]
            wv = w_v[pl.ds(j * _LANES, _LANES)]
            res_v[pl.ds(c + j * _LANES, _LANES)] = cnt * wv
        pltpu.sync_copy(res_v.at[pl.ds(0, c)], sums_hbm)
        pltpu.sync_copy(res_v.at[pl.ds(c, c)], outw_hbm)


def kernel(inputs, labels, weights):
    n, c = inputs.shape
    lab2d = labels.reshape(n // 128, 128)
    br_rows = _BR // 128
    nll = pl.pallas_call(
        _tc_nll_body,
        grid=(n // _BR,),
        in_specs=[
            pl.BlockSpec((_BR, c), lambda i: (i, 0)),
            pl.BlockSpec((br_rows, 128), lambda i: (i, 0)),
        ],
        out_specs=pl.BlockSpec((br_rows, 128), lambda i: (i, 0)),
        out_shape=jax.ShapeDtypeStruct((n // 128, 128), jnp.float32),
    )(inputs, lab2d)

    rows = (n // 128) // _SUBCORES
    mesh = plsc.VectorSubcoreMesh(
        core_axis_name="c", subcore_axis_name="s", num_cores=1)
    sum_by_class, out_weights = pl.kernel(
        functools.partial(_sc_groupby_body, c, rows),
        out_type=(jax.ShapeDtypeStruct((c,), jnp.float32),
                  jax.ShapeDtypeStruct((c,), jnp.float32)),
        mesh=mesh,
        scratch_types=[
            pltpu.VMEM((rows, 128), jnp.int32),
            pltpu.VMEM((rows, 128), jnp.float32),
            pltpu.VMEM((2 * c,), jnp.float32),
            pltpu.VMEM((_SUBCORES, 2 * c), jnp.float32),
            pltpu.VMEM((2 * c,), jnp.float32),
            pltpu.VMEM((c,), jnp.float32),
            pltpu.VMEM_SHARED((_SUBCORES, 2 * c), jnp.float32),
        ],
        compiler_params=pltpu.CompilerParams(needs_layout_passes=False),
    )(lab2d, nll, weights)
    return (sum_by_class, out_weights)
